# baseline (speedup 1.0000x reference)
import functools
import math

import numpy as np
import jax
import jax.numpy as jnp
from jax import lax
from jax.experimental import pallas as pl
from jax.experimental.pallas import tpu as pltpu
from jax.experimental.pallas import tpu_sc as plsc

DIM = 1024
SEQ = 2048
BATCH = 4
SCALE = math.sqrt(DIM)

NC, NS, L = 2, 16, 16
NW = NC * NS
PPW = SEQ // NW
CHP = 8
CHR = CHP * BATCH
NCHUNK = PPW // CHP
RPW = NCHUNK * CHR
VPR = DIM // L
NBUF = 3


def _pos_enc() -> np.ndarray:
    pos = np.arange(SEQ, dtype=np.float64)[:, None]
    idx = np.arange(0, DIM, 2, dtype=np.float64)[None, :]
    angle = pos / (10000.0 ** (idx / DIM))
    pe = np.zeros((SEQ, DIM), dtype=np.float32)
    pe[:, 0::2] = np.sin(angle)
    pe[:, 1::2] = np.cos(angle)
    return pe


_PE = _pos_enc()


def _emb_body(x_hbm, tab_hbm, pe_hbm, out_hbm,
              idx_v, buf, pe_v, gs0, gs1, gs2, ps0, ps1, ps2, os0, os1, os2):
    gsem = (gs0, gs1, gs2)
    psem = (ps0, ps1, ps2)
    osem = (os0, os1, os2)
    wid = lax.axis_index("s") * NC + lax.axis_index("c")
    p0 = wid * PPW

    pltpu.sync_copy(x_hbm.at[wid], idx_v)

    def start_chunk(j):
        slot = j % NBUF
        g = pltpu.async_copy(tab_hbm.at[idx_v.at[j]],
                             buf.at[slot], gsem[slot])
        p = pltpu.async_copy(
            pe_hbm.at[pl.ds(p0 + j * CHP, CHP)],
            pe_v.at[slot], psem[slot])
        return g, p

    def store_chunk(j):
        slot = j % NBUF
        cps = []
        for b in range(BATCH):
            cps.append(pltpu.async_copy(
                buf.at[slot, pl.ds(b * CHP, CHP)],
                out_hbm.at[pl.ds(b * SEQ + p0 + j * CHP, CHP)],
                osem[slot]))
        return cps

    def compute_chunk(j):
        slot = j % NBUF

        @plsc.parallel_loop(0, CHP * VPR, unroll=4)
        def _body(i):
            p = lax.shift_right_logical(i, 6)
            c = lax.bitwise_and(i, VPR - 1)
            sl = pl.ds(c * L, L)
            pv = pe_v[slot, p, sl]
            for b in range(BATCH):
                r = b * CHP + p
                buf[slot, r, sl] = buf[slot, r, sl] * SCALE + pv

    pending_in = [None] * NBUF
    pending_out = [None] * NBUF
    pending_in[0] = start_chunk(0)
    pending_in[1] = start_chunk(1)
    for j in range(NCHUNK):
        slot = j % NBUF
        g, p = pending_in[slot]
        g.wait()
        p.wait()
        compute_chunk(j)
        pending_out[slot] = store_chunk(j)
        nxt = j + 2
        if nxt < NCHUNK:
            ns = nxt % NBUF
            if pending_out[ns] is not None:
                for cp in pending_out[ns]:
                    cp.wait()
                pending_out[ns] = None
            pending_in[ns] = start_chunk(nxt)
    for slot in range(NBUF):
        if pending_out[slot] is not None:
            for cp in pending_out[slot]:
                cp.wait()


@jax.jit
def kernel(x, emb_table):
    x4 = x.reshape(BATCH, NW, NCHUNK, CHP).transpose(1, 2, 0, 3)
    xf = x4.reshape(NW, NCHUNK, CHR)
    mesh = plsc.VectorSubcoreMesh(core_axis_name="c", subcore_axis_name="s")
    run = functools.partial(
        pl.kernel,
        out_type=jax.ShapeDtypeStruct((BATCH * SEQ, DIM), jnp.float32),
        mesh=mesh,
        scratch_types=[
            pltpu.VMEM((NCHUNK, CHR), jnp.int32),
            pltpu.VMEM((NBUF, CHR, DIM), jnp.float32),
            pltpu.VMEM((NBUF, CHP, DIM), jnp.float32),
        ] + [pltpu.SemaphoreType.DMA] * (3 * NBUF),
    )(_emb_body)
    out = run(xf, emb_table, _PE)
    return out.reshape(BATCH, SEQ, DIM)
